# baseline (speedup 1.0000x reference)
import functools
import math

import numpy as np
import jax
import jax.numpy as jnp
from jax import lax
from jax.experimental import pallas as pl
from jax.experimental.pallas import tpu as pltpu
from jax.experimental.pallas import tpu_sc as plsc

L_SEQ = 50
BATCH = 4096
NE = 460
POS = 26
OUT_D = 512
N_AA = 27
N_MOD = 8
N_KEY = N_AA * N_MOD
N_PAIR = 51 * 51
PAIR_W = 128
ROWS = L_SEQ * BATCH
SQRT_NE = math.sqrt(float(NE))
REP = 32


def _make_pe() -> np.ndarray:
    max_len, dims = 128, POS
    position = np.arange(0, max_len, dtype=np.float32)[:, None]
    div_term_enum = np.arange(0, dims, 2, dtype=np.float32)
    div_term_denom = -math.log(10000.0) / dims + 1
    div_term = np.exp(div_term_enum * div_term_denom)
    pe = np.zeros((max_len, dims), dtype=np.float32)
    pe[:, 0::2] = np.sin(position * div_term)
    pe[:, 1::2] = np.cos(position * div_term)
    pe[0, :] = 0.0
    return pe


def _make_pairs() -> np.ndarray:
    pe51 = _make_pe()[:51]
    pairs = np.zeros((51, 51, PAIR_W), dtype=np.float32)
    pairs[:, :, 76:76 + POS] = pe51[:, None, :]
    pairs[:, :, 76 + POS:] = pe51[None, :, :]
    return np.tile(pairs.reshape(N_PAIR, PAIR_W), (REP, 1))


_PAIRS = _make_pairs()


def _prep_body(src_ref, mods_ref, aa_ref, modt_ref, ks_ref, kp_ref, sum_ref):
    srcf = src_ref[...].astype(jnp.float32)
    modsf = mods_ref[...].astype(jnp.float32)
    ks_ref[...] = jnp.transpose(srcf * 8.0 + modsf).astype(jnp.int32)

    mask_t = jnp.transpose(jnp.where(srcf != 0.0, 1.0, 0.0))
    r_io = lax.broadcasted_iota(jnp.int32, (L_SEQ, L_SEQ), 0)
    c_io = lax.broadcasted_iota(jnp.int32, (L_SEQ, L_SEQ), 1)
    lower = jnp.where(r_io >= c_io, 1.0, 0.0)
    fw_t = jnp.dot(lower, mask_t, preferred_element_type=jnp.float32)
    tot_t = fw_t[L_SEQ - 1:L_SEQ, :]
    rev_t = tot_t - fw_t + mask_t
    kp_ref[...] = (fw_t * 51.0 + rev_t).astype(jnp.int32)

    fused = aa_ref[...][:, None, :] + modt_ref[...][None, :, :]
    fused = fused.reshape(N_KEY, NE) * SQRT_NE
    for rep in range(REP):
        sum_ref[pl.ds(rep * N_KEY, N_KEY), 0:NE] = fused
        sum_ref[pl.ds(rep * N_KEY, N_KEY), NE:OUT_D] = jnp.zeros(
            (N_KEY, OUT_D - NE), jnp.float32)


def _prep(src, mods, aa_table, mod_table):
    return pl.pallas_call(
        _prep_body,
        out_shape=(
            jax.ShapeDtypeStruct((L_SEQ, BATCH), jnp.int32),
            jax.ShapeDtypeStruct((L_SEQ, BATCH), jnp.int32),
            jax.ShapeDtypeStruct((REP * N_KEY, OUT_D), jnp.float32),
        ),
    )(src, mods, aa_table, mod_table)


def _sc_info():
    try:
        info = plsc.get_sparse_core_info()
        return info.num_cores, info.num_subcores
    except Exception:
        return 2, 16


NC, NS = _sc_info()
NW = NC * NS
RPW = ROWS // NW
CHUNK = 40
SLOTS = 4
NCHUNK = RPW // CHUNK
GROUPS = NCHUNK // SLOTS


def _sc_body(sum_hbm, pairs_hbm, ks_hbm, kp_hbm, out_hbm,
             ksa_v, kpa_v, a_vs, p_vs, sa_s, sb_s, sw_s):
    wid = lax.axis_index("s") * NC + lax.axis_index("c")
    base = wid * RPW
    pltpu.sync_copy(ks_hbm.at[pl.ds(base, RPW)], ksa_v)
    pltpu.sync_copy(kp_hbm.at[pl.ds(base, RPW)], kpa_v)
    ks_off = (lax.rem(wid, REP) * N_KEY).astype(jnp.int32)
    kp_off = (lax.rem(wid, REP) * N_PAIR).astype(jnp.int32)

    def shift_keys(t, carry):
        ksa_v[pl.ds(t * 16, 16)] = ksa_v[pl.ds(t * 16, 16)] + ks_off
        kpa_v[pl.ds(t * 16, 16)] = kpa_v[pl.ds(t * 16, 16)] + kp_off
        return carry

    lax.fori_loop(0, RPW // 16, shift_keys, 0)

    def group(i, carry):
        g0 = i * SLOTS
        gathers = []
        for s in range(SLOTS):
            off = (g0 + s) * CHUNK
            ca = pltpu.async_copy(
                sum_hbm.at[ksa_v.at[pl.ds(off, CHUNK)]], a_vs[s], sa_s[s])
            cb = pltpu.async_copy(
                pairs_hbm.at[kpa_v.at[pl.ds(off, CHUNK)]], p_vs[s], sb_s[s])
            gathers.append((ca, cb))
        writes = []
        for s in range(SLOTS):
            ca, cb = gathers[s]
            ca.wait()
            cb.wait()
            a_v, p_v = a_vs[s], p_vs[s]

            def addrow(r, c2, a_v=a_v, p_v=p_v):
                a_v[r, pl.ds(448, 16)] = (
                    a_v[r, pl.ds(448, 16)] + p_v[r, pl.ds(64, 16)])
                for j in (1, 2, 3):
                    a_v[r, pl.ds(448 + 16 * j, 16)] = p_v[r, pl.ds(64 + 16 * j, 16)]
                return c2

            lax.fori_loop(0, CHUNK, addrow, 0)
            writes.append(pltpu.async_copy(
                a_v, out_hbm.at[pl.ds(base + (g0 + s) * CHUNK, CHUNK)], sw_s[s]))
        for w in writes:
            w.wait()
        return carry

    lax.fori_loop(0, GROUPS, group, 0)


@functools.partial(jax.jit, static_argnums=())
def _run_sc(sum512, pairs, ks, kp):
    mesh = plsc.VectorSubcoreMesh(core_axis_name="c", subcore_axis_name="s",
                                  num_cores=NC, num_subcores=NS)

    def body(sum_hbm, pairs_hbm, ks_hbm, kp_hbm, out_hbm, *scratch):
        ksa_v, kpa_v = scratch[0], scratch[1]
        a_vs = scratch[2:2 + SLOTS]
        p_vs = scratch[2 + SLOTS:2 + 2 * SLOTS]
        sa_s = scratch[2 + 2 * SLOTS:2 + 3 * SLOTS]
        sb_s = scratch[2 + 3 * SLOTS:2 + 4 * SLOTS]
        sw_s = scratch[2 + 4 * SLOTS:2 + 5 * SLOTS]
        _sc_body(sum_hbm, pairs_hbm, ks_hbm, kp_hbm, out_hbm,
                 ksa_v, kpa_v, a_vs, p_vs, sa_s, sb_s, sw_s)

    f = pl.kernel(
        body,
        out_type=jax.ShapeDtypeStruct((ROWS, OUT_D), jnp.float32),
        mesh=mesh,
        scratch_types=(
            [pltpu.VMEM((RPW,), jnp.int32), pltpu.VMEM((RPW,), jnp.int32)]
            + [pltpu.VMEM((CHUNK, OUT_D), jnp.float32) for _ in range(SLOTS)]
            + [pltpu.VMEM((CHUNK, PAIR_W), jnp.float32) for _ in range(SLOTS)]
            + [pltpu.SemaphoreType.DMA for _ in range(3 * SLOTS)]
        ),
    )
    return f(sum512, pairs, ks, kp)


def kernel(src, mods, aa_table, mod_table):
    src = src.astype(jnp.int32)
    mods = mods.astype(jnp.int32)
    ks, kp, sum512 = _prep(src, mods, aa_table, mod_table)
    pairs = jnp.asarray(_PAIRS)
    out = _run_sc(sum512, pairs, ks.reshape(ROWS), kp.reshape(ROWS))
    return out.reshape(L_SEQ, BATCH, OUT_D)


# speedup vs baseline: 1.0366x; 1.0366x over previous
import functools
import math

import numpy as np
import jax
import jax.numpy as jnp
from jax import lax
from jax.experimental import pallas as pl
from jax.experimental.pallas import tpu as pltpu
from jax.experimental.pallas import tpu_sc as plsc

L_SEQ = 50
BATCH = 4096
NE = 460
POS = 26
OUT_D = 512
N_AA = 27
N_MOD = 8
N_KEY = N_AA * N_MOD
N_PAIR = 51 * 51
PAIR_W = 128
ROWS = L_SEQ * BATCH
SQRT_NE = math.sqrt(float(NE))
REP = 16


def _make_pe() -> np.ndarray:
    max_len, dims = 128, POS
    position = np.arange(0, max_len, dtype=np.float32)[:, None]
    div_term_enum = np.arange(0, dims, 2, dtype=np.float32)
    div_term_denom = -math.log(10000.0) / dims + 1
    div_term = np.exp(div_term_enum * div_term_denom)
    pe = np.zeros((max_len, dims), dtype=np.float32)
    pe[:, 0::2] = np.sin(position * div_term)
    pe[:, 1::2] = np.cos(position * div_term)
    pe[0, :] = 0.0
    return pe


def _make_pairs() -> np.ndarray:
    pe51 = _make_pe()[:51]
    pairs = np.zeros((51, 51, PAIR_W), dtype=np.float32)
    pairs[:, :, 76:76 + POS] = pe51[:, None, :]
    pairs[:, :, 76 + POS:] = pe51[None, :, :]
    return np.tile(pairs.reshape(N_PAIR, PAIR_W), (REP, 1))


_PAIRS = _make_pairs()


def _prep_body(src_ref, mods_ref, aa_ref, modt_ref, ks_ref, kp_ref, sum_ref):
    srcf = src_ref[...].astype(jnp.float32)
    modsf = mods_ref[...].astype(jnp.float32)
    ks_ref[...] = jnp.transpose(srcf * 8.0 + modsf).astype(jnp.int32)

    mask_t = jnp.transpose(jnp.where(srcf != 0.0, 1.0, 0.0))
    r_io = lax.broadcasted_iota(jnp.int32, (L_SEQ, L_SEQ), 0)
    c_io = lax.broadcasted_iota(jnp.int32, (L_SEQ, L_SEQ), 1)
    lower = jnp.where(r_io >= c_io, 1.0, 0.0)
    fw_t = jnp.dot(lower, mask_t, preferred_element_type=jnp.float32)
    tot_t = fw_t[L_SEQ - 1:L_SEQ, :]
    rev_t = tot_t - fw_t + mask_t
    kp_ref[...] = (fw_t * 51.0 + rev_t).astype(jnp.int32)

    fused = aa_ref[...][:, None, :] + modt_ref[...][None, :, :]
    fused = fused.reshape(N_KEY, NE) * SQRT_NE
    for rep in range(REP):
        sum_ref[pl.ds(rep * N_KEY, N_KEY), 0:NE] = fused
        sum_ref[pl.ds(rep * N_KEY, N_KEY), NE:OUT_D] = jnp.zeros(
            (N_KEY, OUT_D - NE), jnp.float32)


def _prep(src, mods, aa_table, mod_table):
    return pl.pallas_call(
        _prep_body,
        out_shape=(
            jax.ShapeDtypeStruct((L_SEQ, BATCH), jnp.int32),
            jax.ShapeDtypeStruct((L_SEQ, BATCH), jnp.int32),
            jax.ShapeDtypeStruct((REP * N_KEY, OUT_D), jnp.float32),
        ),
    )(src, mods, aa_table, mod_table)


def _sc_info():
    try:
        info = plsc.get_sparse_core_info()
        return info.num_cores, info.num_subcores
    except Exception:
        return 2, 16


NC, NS = _sc_info()
NW = NC * NS
RPW = ROWS // NW
CHUNK = 40
SLOTS = 4
NCHUNK = RPW // CHUNK
GROUPS = NCHUNK // SLOTS


def _sc_body(sum_hbm, pairs_hbm, ks_hbm, kp_hbm, out_hbm,
             ksa_v, kpa_v, a_vs, p_vs, sa_s, sb_s, sw_s):
    wid = lax.axis_index("s") * NC + lax.axis_index("c")
    base = wid * RPW
    pltpu.sync_copy(ks_hbm.at[pl.ds(base, RPW)], ksa_v)
    pltpu.sync_copy(kp_hbm.at[pl.ds(base, RPW)], kpa_v)
    ks_off = (lax.rem(wid, REP) * N_KEY).astype(jnp.int32)
    kp_off = (lax.rem(wid, REP) * N_PAIR).astype(jnp.int32)

    def shift_keys(t, carry):
        ksa_v[pl.ds(t * 16, 16)] = ksa_v[pl.ds(t * 16, 16)] + ks_off
        kpa_v[pl.ds(t * 16, 16)] = kpa_v[pl.ds(t * 16, 16)] + kp_off
        return carry

    lax.fori_loop(0, RPW // 16, shift_keys, 0)

    def group(i, carry):
        g0 = i * SLOTS
        gathers = []
        for s in range(SLOTS):
            off = (g0 + s) * CHUNK
            ca = pltpu.async_copy(
                sum_hbm.at[ksa_v.at[pl.ds(off, CHUNK)]], a_vs[s], sa_s[s])
            cb = pltpu.async_copy(
                pairs_hbm.at[kpa_v.at[pl.ds(off, CHUNK)]], p_vs[s], sb_s[s])
            gathers.append((ca, cb))
        writes = []
        for s in range(SLOTS):
            ca, cb = gathers[s]
            ca.wait()
            cb.wait()
            a_v, p_v = a_vs[s], p_vs[s]

            def addrow(r, c2, a_v=a_v, p_v=p_v):
                a_v[r, pl.ds(448, 16)] = (
                    a_v[r, pl.ds(448, 16)] + p_v[r, pl.ds(64, 16)])
                for j in (1, 2, 3):
                    a_v[r, pl.ds(448 + 16 * j, 16)] = p_v[r, pl.ds(64 + 16 * j, 16)]
                return c2

            lax.fori_loop(0, CHUNK, addrow, 0)
            writes.append(pltpu.async_copy(
                a_v, out_hbm.at[pl.ds(base + (g0 + s) * CHUNK, CHUNK)], sw_s[s]))
        for w in writes:
            w.wait()
        return carry

    lax.fori_loop(0, GROUPS, group, 0)


@functools.partial(jax.jit, static_argnums=())
def _run_sc(sum512, pairs, ks, kp):
    mesh = plsc.VectorSubcoreMesh(core_axis_name="c", subcore_axis_name="s",
                                  num_cores=NC, num_subcores=NS)

    def body(sum_hbm, pairs_hbm, ks_hbm, kp_hbm, out_hbm, *scratch):
        ksa_v, kpa_v = scratch[0], scratch[1]
        a_vs = scratch[2:2 + SLOTS]
        p_vs = scratch[2 + SLOTS:2 + 2 * SLOTS]
        sa_s = scratch[2 + 2 * SLOTS:2 + 3 * SLOTS]
        sb_s = scratch[2 + 3 * SLOTS:2 + 4 * SLOTS]
        sw_s = scratch[2 + 4 * SLOTS:2 + 5 * SLOTS]
        _sc_body(sum_hbm, pairs_hbm, ks_hbm, kp_hbm, out_hbm,
                 ksa_v, kpa_v, a_vs, p_vs, sa_s, sb_s, sw_s)

    f = pl.kernel(
        body,
        out_type=jax.ShapeDtypeStruct((ROWS, OUT_D), jnp.float32),
        mesh=mesh,
        scratch_types=(
            [pltpu.VMEM((RPW,), jnp.int32), pltpu.VMEM((RPW,), jnp.int32)]
            + [pltpu.VMEM((CHUNK, OUT_D), jnp.float32) for _ in range(SLOTS)]
            + [pltpu.VMEM((CHUNK, PAIR_W), jnp.float32) for _ in range(SLOTS)]
            + [pltpu.SemaphoreType.DMA for _ in range(3 * SLOTS)]
        ),
    )
    return f(sum512, pairs, ks, kp)


def kernel(src, mods, aa_table, mod_table):
    src = src.astype(jnp.int32)
    mods = mods.astype(jnp.int32)
    ks, kp, sum512 = _prep(src, mods, aa_table, mod_table)
    pairs = jnp.asarray(_PAIRS)
    out = _run_sc(sum512, pairs, ks.reshape(ROWS), kp.reshape(ROWS))
    return out.reshape(L_SEQ, BATCH, OUT_D)
